# Initial kernel scaffold; baseline (speedup 1.0000x reference)
import jax
import jax.numpy as jnp
from jax.experimental import pallas as pl


def kernel(x, edge_index, relations, Wl0, bl0, Wr0, Wl1, bl1, Wr1, Wlin, blin):
    raise NotImplementedError("write your pallas kernel here")


# speedup vs baseline: 3.0289x; 3.0289x over previous
import functools

import jax
import jax.numpy as jnp
from jax import lax
from jax.experimental import pallas as pl
from jax.experimental.pallas import tpu as pltpu
from jax.experimental.pallas import tpu_sc as plsc

_N, _E, _D, _H, _C = 10000, 320000, 128, 128, 41
_NC, _NS = 2, 16
_NW = _NC * _NS
_CH = 128
_NCHUNK = 80
_EPW = _NCHUNK * _CH
_EP = _NW * _EPW
_BLK = 8
_NBLK = _NCHUNK // _BLK
_NP = 10112
_RPT = _NP // _NS

_f32 = jnp.float32

_mesh = plsc.VectorSubcoreMesh(core_axis_name="c", subcore_axis_name="s")


def _cnt_body(dst3, zrows, ones_hbm, pcnt, cnt_sh, didx_blk, ones_v):
    cid = lax.axis_index("c")
    sid = lax.axis_index("s")
    wid = sid * _NC + cid
    rbase = sid * _RPT

    pltpu.sync_copy(zrows, cnt_sh.at[pl.ds(rbase, _RPT)])
    pltpu.sync_copy(ones_hbm, ones_v)
    plsc.subcore_barrier()

    def blk_step(bk, carry):
        row0 = pl.multiple_of(bk * _BLK, _BLK)
        pltpu.sync_copy(dst3.at[wid, pl.ds(row0, _BLK)], didx_blk)
        for b in range(_BLK):
            pltpu.sync_copy(ones_v, cnt_sh.at[didx_blk.at[b]], add=True)
        return carry

    lax.fori_loop(0, _NBLK, blk_step, 0)
    plsc.subcore_barrier()

    pltpu.sync_copy(cnt_sh.at[pl.ds(rbase, _RPT)],
                    pcnt.at[cid, pl.ds(rbase, _RPT)])


def _seg_body(feat, src3, dst3, zrows, psum,
              acc_sh, idx_blk, didx_blk, rows_v, sem):
    cid = lax.axis_index("c")
    sid = lax.axis_index("s")
    wid = sid * _NC + cid
    rbase = sid * _RPT

    pltpu.sync_copy(zrows, acc_sh.at[pl.ds(rbase, _RPT)])
    plsc.subcore_barrier()

    def blk_step(bk, carry):
        row0 = pl.multiple_of(bk * _BLK, _BLK)
        pltpu.sync_copy(src3.at[wid, pl.ds(row0, _BLK)], idx_blk)
        pltpu.sync_copy(dst3.at[wid, pl.ds(row0, _BLK)], didx_blk)
        for b in range(_BLK):
            pltpu.async_copy(feat.at[idx_blk.at[b]], rows_v, sem).wait()
            pltpu.sync_copy(rows_v, acc_sh.at[didx_blk.at[b]], add=True)
        return carry

    lax.fori_loop(0, _NBLK, blk_step, 0)
    plsc.subcore_barrier()

    pltpu.sync_copy(acc_sh.at[pl.ds(rbase, _RPT)],
                    psum.at[cid, pl.ds(rbase, _RPT)])


_cnt_pass = pl.kernel(
    _cnt_body,
    out_type=jax.ShapeDtypeStruct((_NC, _NP, _D), _f32),
    mesh=_mesh,
    scratch_types=[
        pltpu.VMEM_SHARED((_NP, _D), _f32),
        pltpu.VMEM((_BLK, _CH), jnp.int32),
        pltpu.VMEM((_CH, _D), _f32),
    ],
)

_seg_sum = pl.kernel(
    _seg_body,
    out_type=jax.ShapeDtypeStruct((_NC, _NP, _D), _f32),
    mesh=_mesh,
    scratch_types=[
        pltpu.VMEM_SHARED((_NP, _D), _f32),
        pltpu.VMEM((_BLK, _CH), jnp.int32),
        pltpu.VMEM((_BLK, _CH), jnp.int32),
        pltpu.VMEM((_CH, _D), _f32),
        pltpu.SemaphoreType.DMA,
    ],
)


def _dot(a, b):
    return jnp.dot(a, b, precision=lax.Precision.HIGHEST,
                   preferred_element_type=_f32)


def _layer1_body(ps, pc, xin, wl, bl, wr, out):
    p = ps[...]
    c = pc[...]
    cnt = c[0, :, 0:1] + c[1, :, 0:1]
    mean = (p[0] + p[1]) / jnp.maximum(cnt, 1.0)
    h = _dot(mean, wl[...]) + bl[...] + _dot(xin[...], wr[...])
    out[...] = jnp.maximum(h, 0.0)


def _layer2_body(ps, pc, hin, wl, bl, wr, wlin, blin, out):
    p = ps[...]
    c = pc[...]
    cnt = c[0, :, 0:1] + c[1, :, 0:1]
    mean = (p[0] + p[1]) / jnp.maximum(cnt, 1.0)
    h1 = _dot(mean, wl[...]) + bl[...] + _dot(hin[...], wr[...])
    out[...] = _dot(h1, wlin[...]) + blin[...]


_BS = 2000
_GRID = (_N // _BS,)


def _layer1(psum, pcnt, x, Wl, bl, Wr):
    return pl.pallas_call(
        _layer1_body,
        grid=_GRID,
        in_specs=[
            pl.BlockSpec((_NC, _BS, _D), lambda i: (0, i, 0)),
            pl.BlockSpec((_NC, _BS, _D), lambda i: (0, i, 0)),
            pl.BlockSpec((_BS, _D), lambda i: (i, 0)),
            pl.BlockSpec((_D, _H), lambda i: (0, 0)),
            pl.BlockSpec((1, _H), lambda i: (0, 0)),
            pl.BlockSpec((_D, _H), lambda i: (0, 0)),
        ],
        out_specs=pl.BlockSpec((_BS, _H), lambda i: (i, 0)),
        out_shape=jax.ShapeDtypeStruct((_N, _H), _f32),
    )(psum, pcnt, x, Wl, bl, Wr)


def _layer2(psum, pcnt, h, Wl, bl, Wr, Wlin, blin):
    return pl.pallas_call(
        _layer2_body,
        grid=_GRID,
        in_specs=[
            pl.BlockSpec((_NC, _BS, _H), lambda i: (0, i, 0)),
            pl.BlockSpec((_NC, _BS, _D), lambda i: (0, i, 0)),
            pl.BlockSpec((_BS, _H), lambda i: (i, 0)),
            pl.BlockSpec((_H, _H), lambda i: (0, 0)),
            pl.BlockSpec((1, _H), lambda i: (0, 0)),
            pl.BlockSpec((_H, _H), lambda i: (0, 0)),
            pl.BlockSpec((_H, _C), lambda i: (0, 0)),
            pl.BlockSpec((1, _C), lambda i: (0, 0)),
        ],
        out_specs=pl.BlockSpec((_BS, _C), lambda i: (i, 0)),
        out_shape=jax.ShapeDtypeStruct((_N, _C), _f32),
    )(psum, pcnt, h, Wl, bl, Wr, Wlin, blin)


def kernel(x, edge_index, relations, Wl0, bl0, Wr0, Wl1, bl1, Wr1, Wlin, blin):
    del relations
    npad = _EP - _E
    src = jnp.concatenate([edge_index[0], jnp.zeros((npad,), jnp.int32)])
    dst = jnp.concatenate([edge_index[1],
                           jnp.full((npad,), _NP - 1, jnp.int32)])
    src3 = src.reshape(_NW, _NCHUNK, _CH)
    dst3 = dst.reshape(_NW, _NCHUNK, _CH)
    zrows = jnp.zeros((_RPT, _D), _f32)
    ones = jnp.ones((_CH, _D), _f32)

    pcnt = _cnt_pass(dst3, zrows, ones)
    psum0 = _seg_sum(x, src3, dst3, zrows)
    h = _layer1(psum0, pcnt, x, Wl0, bl0.reshape(1, _H), Wr0)
    psum1 = _seg_sum(h, src3, dst3, zrows)
    out = _layer2(psum1, pcnt, h, Wl1, bl1.reshape(1, _H), Wr1,
                  Wlin, blin.reshape(1, _C))
    return out


# speedup vs baseline: 3.2242x; 1.0644x over previous
import functools

import jax
import jax.numpy as jnp
from jax import lax
from jax.experimental import pallas as pl
from jax.experimental.pallas import tpu as pltpu
from jax.experimental.pallas import tpu_sc as plsc

_N, _E, _D, _H, _C = 10000, 320000, 128, 128, 41
_NC, _NS = 2, 16
_NW = _NC * _NS
_CH = 128
_NCHUNK = 80
_EPW = _NCHUNK * _CH
_EP = _NW * _EPW
_BLK = 8
_NBLK = _NCHUNK // _BLK
_NP = 10112
_RPT = _NP // _NS

_f32 = jnp.float32

_mesh = plsc.VectorSubcoreMesh(core_axis_name="c", subcore_axis_name="s")


def _cnt_body(dst3, zrows, ones_hbm, pcnt, cnt_sh, didx_blk, ones_v, ssem):
    cid = lax.axis_index("c")
    sid = lax.axis_index("s")
    wid = sid * _NC + cid
    rbase = sid * _RPT

    pltpu.sync_copy(zrows, cnt_sh.at[pl.ds(rbase, _RPT)])
    pltpu.sync_copy(ones_hbm, ones_v)
    plsc.subcore_barrier()

    def blk_step(bk, carry):
        row0 = pl.multiple_of(bk * _BLK, _BLK)
        pltpu.sync_copy(dst3.at[wid, pl.ds(row0, _BLK)], didx_blk)
        hs = [pltpu.async_copy(ones_v, cnt_sh.at[didx_blk.at[b]], ssem,
                               add=True)
              for b in range(_BLK)]
        for h in hs:
            h.wait()
        return carry

    lax.fori_loop(0, _NBLK, blk_step, 0)
    plsc.subcore_barrier()

    pltpu.sync_copy(cnt_sh.at[pl.ds(rbase, _RPT)],
                    pcnt.at[cid, pl.ds(rbase, _RPT)])


def _seg_body(feat, src3, dst3, zrows, psum,
              acc_sh, idx_blk, didx_blk, rows2, gsem, ssem):
    cid = lax.axis_index("c")
    sid = lax.axis_index("s")
    wid = sid * _NC + cid
    rbase = sid * _RPT

    pltpu.sync_copy(zrows, acc_sh.at[pl.ds(rbase, _RPT)])
    plsc.subcore_barrier()

    def blk_step(bk, carry):
        row0 = pl.multiple_of(bk * _BLK, _BLK)
        pltpu.sync_copy(src3.at[wid, pl.ds(row0, _BLK)], idx_blk)
        pltpu.sync_copy(dst3.at[wid, pl.ds(row0, _BLK)], didx_blk)
        gh = {0: pltpu.async_copy(feat.at[idx_blk.at[0]], rows2.at[0], gsem)}
        sh = {}
        for b in range(_BLK):
            gh[b].wait()
            if b >= 1:
                sh[b - 1].wait()
            if b < _BLK - 1:
                gh[b + 1] = pltpu.async_copy(feat.at[idx_blk.at[b + 1]],
                                             rows2.at[(b + 1) % 2], gsem)
            sh[b] = pltpu.async_copy(rows2.at[b % 2],
                                     acc_sh.at[didx_blk.at[b]], ssem,
                                     add=True)
        sh[_BLK - 1].wait()
        return carry

    lax.fori_loop(0, _NBLK, blk_step, 0)
    plsc.subcore_barrier()

    pltpu.sync_copy(acc_sh.at[pl.ds(rbase, _RPT)],
                    psum.at[cid, pl.ds(rbase, _RPT)])


_cnt_pass = pl.kernel(
    _cnt_body,
    out_type=jax.ShapeDtypeStruct((_NC, _NP, _D), _f32),
    mesh=_mesh,
    scratch_types=[
        pltpu.VMEM_SHARED((_NP, _D), _f32),
        pltpu.VMEM((_BLK, _CH), jnp.int32),
        pltpu.VMEM((_CH, _D), _f32),
        pltpu.SemaphoreType.DMA,
    ],
)

_seg_sum = pl.kernel(
    _seg_body,
    out_type=jax.ShapeDtypeStruct((_NC, _NP, _D), _f32),
    mesh=_mesh,
    scratch_types=[
        pltpu.VMEM_SHARED((_NP, _D), _f32),
        pltpu.VMEM((_BLK, _CH), jnp.int32),
        pltpu.VMEM((_BLK, _CH), jnp.int32),
        pltpu.VMEM((2, _CH, _D), _f32),
        pltpu.SemaphoreType.DMA,
        pltpu.SemaphoreType.DMA,
    ],
)


def _dot(a, b):
    return jnp.dot(a, b, precision=lax.Precision.HIGHEST,
                   preferred_element_type=_f32)


def _layer1_body(ps, pc, xin, wl, bl, wr, out):
    p = ps[...]
    c = pc[...]
    cnt = c[0, :, 0:1] + c[1, :, 0:1]
    mean = (p[0] + p[1]) / jnp.maximum(cnt, 1.0)
    h = _dot(mean, wl[...]) + bl[...] + _dot(xin[...], wr[...])
    out[...] = jnp.maximum(h, 0.0)


def _layer2_body(ps, pc, hin, wl, bl, wr, wlin, blin, out):
    p = ps[...]
    c = pc[...]
    cnt = c[0, :, 0:1] + c[1, :, 0:1]
    mean = (p[0] + p[1]) / jnp.maximum(cnt, 1.0)
    h1 = _dot(mean, wl[...]) + bl[...] + _dot(hin[...], wr[...])
    out[...] = _dot(h1, wlin[...]) + blin[...]


_BS = 2000
_GRID = (_N // _BS,)


def _layer1(psum, pcnt, x, Wl, bl, Wr):
    return pl.pallas_call(
        _layer1_body,
        grid=_GRID,
        in_specs=[
            pl.BlockSpec((_NC, _BS, _D), lambda i: (0, i, 0)),
            pl.BlockSpec((_NC, _BS, _D), lambda i: (0, i, 0)),
            pl.BlockSpec((_BS, _D), lambda i: (i, 0)),
            pl.BlockSpec((_D, _H), lambda i: (0, 0)),
            pl.BlockSpec((1, _H), lambda i: (0, 0)),
            pl.BlockSpec((_D, _H), lambda i: (0, 0)),
        ],
        out_specs=pl.BlockSpec((_BS, _H), lambda i: (i, 0)),
        out_shape=jax.ShapeDtypeStruct((_N, _H), _f32),
    )(psum, pcnt, x, Wl, bl, Wr)


def _layer2(psum, pcnt, h, Wl, bl, Wr, Wlin, blin):
    return pl.pallas_call(
        _layer2_body,
        grid=_GRID,
        in_specs=[
            pl.BlockSpec((_NC, _BS, _H), lambda i: (0, i, 0)),
            pl.BlockSpec((_NC, _BS, _D), lambda i: (0, i, 0)),
            pl.BlockSpec((_BS, _H), lambda i: (i, 0)),
            pl.BlockSpec((_H, _H), lambda i: (0, 0)),
            pl.BlockSpec((1, _H), lambda i: (0, 0)),
            pl.BlockSpec((_H, _H), lambda i: (0, 0)),
            pl.BlockSpec((_H, _C), lambda i: (0, 0)),
            pl.BlockSpec((1, _C), lambda i: (0, 0)),
        ],
        out_specs=pl.BlockSpec((_BS, _C), lambda i: (i, 0)),
        out_shape=jax.ShapeDtypeStruct((_N, _C), _f32),
    )(psum, pcnt, h, Wl, bl, Wr, Wlin, blin)


def kernel(x, edge_index, relations, Wl0, bl0, Wr0, Wl1, bl1, Wr1, Wlin, blin):
    del relations
    npad = _EP - _E
    src = jnp.concatenate([edge_index[0], jnp.zeros((npad,), jnp.int32)])
    dst = jnp.concatenate([edge_index[1],
                           jnp.full((npad,), _NP - 1, jnp.int32)])
    src3 = src.reshape(_NW, _NCHUNK, _CH)
    dst3 = dst.reshape(_NW, _NCHUNK, _CH)
    zrows = jnp.zeros((_RPT, _D), _f32)
    ones = jnp.ones((_CH, _D), _f32)

    pcnt = _cnt_pass(dst3, zrows, ones)
    psum0 = _seg_sum(x, src3, dst3, zrows)
    h = _layer1(psum0, pcnt, x, Wl0, bl0.reshape(1, _H), Wr0)
    psum1 = _seg_sum(h, src3, dst3, zrows)
    out = _layer2(psum1, pcnt, h, Wl1, bl1.reshape(1, _H), Wr1,
                  Wlin, blin.reshape(1, _C))
    return out


# speedup vs baseline: 3.6896x; 1.1444x over previous
import functools

import jax
import jax.numpy as jnp
from jax import lax
from jax.experimental import pallas as pl
from jax.experimental.pallas import tpu as pltpu
from jax.experimental.pallas import tpu_sc as plsc

_N, _E, _D, _H, _C = 10000, 320000, 128, 128, 41
_NC, _NS = 2, 16
_NW = _NC * _NS
_CH = 128
_NCHUNK = 80
_EPW = _NCHUNK * _CH
_EP = _NW * _EPW
_BLK = 8
_NBLK = _NCHUNK // _BLK
_NCH_A = 120
_NCH_B = 2 * _NCHUNK - _NCH_A
_NP = 10112
_RPT = _NP // _NS

_f32 = jnp.float32

_mesh = plsc.VectorSubcoreMesh(core_axis_name="c", subcore_axis_name="s")


def _cnt_body(dst3, zrows, ones_hbm, pcnt, cnt_sh, didx_blk, ones_v, ssem):
    cid = lax.axis_index("c")
    sid = lax.axis_index("s")
    wid = sid * _NC + cid
    rbase = sid * _RPT

    pltpu.sync_copy(zrows, cnt_sh.at[pl.ds(rbase, _RPT)])
    pltpu.sync_copy(ones_hbm, ones_v)
    plsc.subcore_barrier()

    def blk_step(bk, carry):
        row0 = pl.multiple_of(bk * _BLK, _BLK)
        pltpu.sync_copy(dst3.at[wid, pl.ds(row0, _BLK)], didx_blk)
        hs = [pltpu.async_copy(ones_v, cnt_sh.at[didx_blk.at[b]], ssem,
                               add=True)
              for b in range(_BLK)]
        for h in hs:
            h.wait()
        return carry

    lax.fori_loop(0, _NBLK, blk_step, 0)
    plsc.subcore_barrier()

    pltpu.sync_copy(cnt_sh.at[pl.ds(rbase, _RPT)],
                    pcnt.at[cid, pl.ds(rbase, _RPT)])


def _seg_body(feat, srcA, dstA, srcB, dstB, zrows, psum,
              acc_sh, idx_blk, didx_blk, rows2, gsem, ssem):
    cid = lax.axis_index("c")
    sid = lax.axis_index("s")
    rbase = sid * _RPT

    pltpu.sync_copy(zrows, acc_sh.at[pl.ds(rbase, _RPT)])
    plsc.subcore_barrier()

    def blk_step(src3, dst3, bk):
        row0 = pl.multiple_of(bk * _BLK, _BLK)
        pltpu.sync_copy(src3.at[sid, pl.ds(row0, _BLK)], idx_blk)
        pltpu.sync_copy(dst3.at[sid, pl.ds(row0, _BLK)], didx_blk)
        gh = {0: pltpu.async_copy(feat.at[idx_blk.at[0]], rows2.at[0], gsem)}
        sh = {}
        for b in range(_BLK):
            gh[b].wait()
            if b >= 1:
                sh[b - 1].wait()
            if b < _BLK - 1:
                gh[b + 1] = pltpu.async_copy(feat.at[idx_blk.at[b + 1]],
                                             rows2.at[(b + 1) % 2], gsem)
            sh[b] = pltpu.async_copy(rows2.at[b % 2],
                                     acc_sh.at[didx_blk.at[b]], ssem,
                                     add=True)
        sh[_BLK - 1].wait()

    @pl.when(cid == 0)
    def _():
        lax.fori_loop(0, _NCH_A // _BLK,
                      lambda bk, c: (blk_step(srcA, dstA, bk), c)[1], 0)

    @pl.when(cid == 1)
    def _():
        lax.fori_loop(0, _NCH_B // _BLK,
                      lambda bk, c: (blk_step(srcB, dstB, bk), c)[1], 0)

    plsc.subcore_barrier()

    pltpu.sync_copy(acc_sh.at[pl.ds(rbase, _RPT)],
                    psum.at[cid, pl.ds(rbase, _RPT)])


_cnt_pass = pl.kernel(
    _cnt_body,
    out_type=jax.ShapeDtypeStruct((_NC, _NP, _D), _f32),
    mesh=_mesh,
    scratch_types=[
        pltpu.VMEM_SHARED((_NP, _D), _f32),
        pltpu.VMEM((_BLK, _CH), jnp.int32),
        pltpu.VMEM((_CH, _D), _f32),
        pltpu.SemaphoreType.DMA,
    ],
)

_seg_sum = pl.kernel(
    _seg_body,
    out_type=jax.ShapeDtypeStruct((_NC, _NP, _D), _f32),
    mesh=_mesh,
    scratch_types=[
        pltpu.VMEM_SHARED((_NP, _D), _f32),
        pltpu.VMEM((_BLK, _CH), jnp.int32),
        pltpu.VMEM((_BLK, _CH), jnp.int32),
        pltpu.VMEM((2, _CH, _D), _f32),
        pltpu.SemaphoreType.DMA,
        pltpu.SemaphoreType.DMA,
    ],
)

_EA = _NS * _NCH_A * _CH


def _dot(a, b):
    return jnp.dot(a, b, precision=lax.Precision.HIGHEST,
                   preferred_element_type=_f32)


def _layer1_body(ps, pc, xin, wl, bl, wr, out):
    p = ps[...]
    c = pc[...]
    cnt = c[0, :, 0:1] + c[1, :, 0:1]
    mean = (p[0] + p[1]) / jnp.maximum(cnt, 1.0)
    h = _dot(mean, wl[...]) + bl[...] + _dot(xin[...], wr[...])
    out[...] = jnp.maximum(h, 0.0)


def _layer2_body(ps, pc, hin, wl, bl, wr, wlin, blin, out):
    p = ps[...]
    c = pc[...]
    cnt = c[0, :, 0:1] + c[1, :, 0:1]
    mean = (p[0] + p[1]) / jnp.maximum(cnt, 1.0)
    h1 = _dot(mean, wl[...]) + bl[...] + _dot(hin[...], wr[...])
    out[...] = _dot(h1, wlin[...]) + blin[...]


_BS = 2000
_GRID = (_N // _BS,)


def _layer1(psum, pcnt, x, Wl, bl, Wr):
    return pl.pallas_call(
        _layer1_body,
        grid=_GRID,
        in_specs=[
            pl.BlockSpec((_NC, _BS, _D), lambda i: (0, i, 0)),
            pl.BlockSpec((_NC, _BS, _D), lambda i: (0, i, 0)),
            pl.BlockSpec((_BS, _D), lambda i: (i, 0)),
            pl.BlockSpec((_D, _H), lambda i: (0, 0)),
            pl.BlockSpec((1, _H), lambda i: (0, 0)),
            pl.BlockSpec((_D, _H), lambda i: (0, 0)),
        ],
        out_specs=pl.BlockSpec((_BS, _H), lambda i: (i, 0)),
        out_shape=jax.ShapeDtypeStruct((_N, _H), _f32),
    )(psum, pcnt, x, Wl, bl, Wr)


def _layer2(psum, pcnt, h, Wl, bl, Wr, Wlin, blin):
    return pl.pallas_call(
        _layer2_body,
        grid=_GRID,
        in_specs=[
            pl.BlockSpec((_NC, _BS, _H), lambda i: (0, i, 0)),
            pl.BlockSpec((_NC, _BS, _D), lambda i: (0, i, 0)),
            pl.BlockSpec((_BS, _H), lambda i: (i, 0)),
            pl.BlockSpec((_H, _H), lambda i: (0, 0)),
            pl.BlockSpec((1, _H), lambda i: (0, 0)),
            pl.BlockSpec((_H, _H), lambda i: (0, 0)),
            pl.BlockSpec((_H, _C), lambda i: (0, 0)),
            pl.BlockSpec((1, _C), lambda i: (0, 0)),
        ],
        out_specs=pl.BlockSpec((_BS, _C), lambda i: (i, 0)),
        out_shape=jax.ShapeDtypeStruct((_N, _C), _f32),
    )(psum, pcnt, h, Wl, bl, Wr, Wlin, blin)


def kernel(x, edge_index, relations, Wl0, bl0, Wr0, Wl1, bl1, Wr1, Wlin, blin):
    del relations
    npad = _EP - _E
    src = jnp.concatenate([edge_index[0], jnp.zeros((npad,), jnp.int32)])
    dst = jnp.concatenate([edge_index[1],
                           jnp.full((npad,), _NP - 1, jnp.int32)])
    src3 = src.reshape(_NW, _NCHUNK, _CH)
    dst3 = dst.reshape(_NW, _NCHUNK, _CH)
    srcA = src[:_EA].reshape(_NS, _NCH_A, _CH)
    dstA = dst[:_EA].reshape(_NS, _NCH_A, _CH)
    srcB = src[_EA:].reshape(_NS, _NCH_B, _CH)
    dstB = dst[_EA:].reshape(_NS, _NCH_B, _CH)
    zrows = jnp.zeros((_RPT, _D), _f32)
    ones = jnp.ones((_CH, _D), _f32)

    pcnt = _cnt_pass(dst3, zrows, ones)
    psum0 = _seg_sum(x, srcA, dstA, srcB, dstB, zrows)
    h = _layer1(psum0, pcnt, x, Wl0, bl0.reshape(1, _H), Wr0)
    psum1 = _seg_sum(h, srcA, dstA, srcB, dstB, zrows)
    out = _layer2(psum1, pcnt, h, Wl1, bl1.reshape(1, _H), Wr1,
                  Wlin, blin.reshape(1, _C))
    return out


# speedup vs baseline: 8.4802x; 2.2984x over previous
import functools

import jax
import jax.numpy as jnp
from jax import lax
from jax.experimental import pallas as pl
from jax.experimental.pallas import tpu as pltpu
from jax.experimental.pallas import tpu_sc as plsc

_N, _E, _D, _H, _C = 10000, 320000, 128, 128, 41
_NC, _NS = 2, 16
_NW = _NC * _NS
_CH = 128
_NCHUNK = 80
_EPW = _NCHUNK * _CH
_EP = _NW * _EPW
_BLK = 8
_NBLK = _NCHUNK // _BLK
_NCH_A = 80
_NCH_B = 2 * _NCHUNK - _NCH_A
_NP = 10112
_RPT = _NP // _NS

_f32 = jnp.float32

_mesh = plsc.VectorSubcoreMesh(core_axis_name="c", subcore_axis_name="s")


def _cnt_body(dst3, zrows, ones_hbm, pcnt, cnt_sh, didx_blk, ones_v, ssem):
    cid = lax.axis_index("c")
    sid = lax.axis_index("s")
    wid = sid * _NC + cid
    rbase = sid * _RPT

    pltpu.sync_copy(zrows, cnt_sh.at[pl.ds(rbase, _RPT)])
    pltpu.sync_copy(ones_hbm, ones_v)
    plsc.subcore_barrier()

    def blk_step(bk, carry):
        row0 = pl.multiple_of(bk * _BLK, _BLK)
        pltpu.sync_copy(dst3.at[wid, pl.ds(row0, _BLK)], didx_blk)
        hs = [pltpu.async_copy(ones_v, cnt_sh.at[didx_blk.at[b]], ssem,
                               add=True)
              for b in range(_BLK)]
        for h in hs:
            h.wait()
        return carry

    lax.fori_loop(0, _NBLK, blk_step, 0)
    plsc.subcore_barrier()

    pltpu.sync_copy(cnt_sh.at[pl.ds(rbase, _RPT)],
                    pcnt.at[cid, pl.ds(rbase, _RPT)])


def _seg_body(feat, srcA, dstA, srcB, dstB, zrows, psum,
              acc_sh, idx_blk, didx_blk, rows2, gsem, ssem):
    cid = lax.axis_index("c")
    sid = lax.axis_index("s")
    rbase = sid * _RPT

    pltpu.sync_copy(zrows, acc_sh.at[pl.ds(rbase, _RPT)])
    plsc.subcore_barrier()

    def blk_step(src3, dst3, bk):
        row0 = pl.multiple_of(bk * _BLK, _BLK)
        pltpu.sync_copy(src3.at[sid, pl.ds(row0, _BLK)], idx_blk)
        pltpu.sync_copy(dst3.at[sid, pl.ds(row0, _BLK)], didx_blk)
        gh = {0: pltpu.async_copy(feat.at[idx_blk.at[0]], rows2.at[0], gsem)}
        sh = {}
        for b in range(_BLK):
            gh[b].wait()
            if b >= 1:
                sh[b - 1].wait()
            if b < _BLK - 1:
                gh[b + 1] = pltpu.async_copy(feat.at[idx_blk.at[b + 1]],
                                             rows2.at[(b + 1) % 2], gsem)
            sh[b] = pltpu.async_copy(rows2.at[b % 2],
                                     acc_sh.at[didx_blk.at[b]], ssem,
                                     add=True)
        sh[_BLK - 1].wait()

    @pl.when(cid == 0)
    def _():
        lax.fori_loop(0, _NCH_A // _BLK,
                      lambda bk, c: (blk_step(srcA, dstA, bk), c)[1], 0)

    @pl.when(cid == 1)
    def _():
        lax.fori_loop(0, _NCH_B // _BLK,
                      lambda bk, c: (blk_step(srcB, dstB, bk), c)[1], 0)

    plsc.subcore_barrier()

    pltpu.sync_copy(acc_sh.at[pl.ds(rbase, _RPT)],
                    psum.at[cid, pl.ds(rbase, _RPT)])


_cnt_pass = pl.kernel(
    _cnt_body,
    out_type=jax.ShapeDtypeStruct((_NC, _NP, _D), _f32),
    mesh=_mesh,
    scratch_types=[
        pltpu.VMEM_SHARED((_NP, _D), _f32),
        pltpu.VMEM((_BLK, _CH), jnp.int32),
        pltpu.VMEM((_CH, _D), _f32),
        pltpu.SemaphoreType.DMA,
    ],
)

_seg_sum = pl.kernel(
    _seg_body,
    out_type=jax.ShapeDtypeStruct((_NC, _NP, _D), _f32),
    mesh=_mesh,
    scratch_types=[
        pltpu.VMEM_SHARED((_NP, _D), _f32),
        pltpu.VMEM((_BLK, _CH), jnp.int32),
        pltpu.VMEM((_BLK, _CH), jnp.int32),
        pltpu.VMEM((2, _CH, _D), _f32),
        pltpu.SemaphoreType.DMA,
        pltpu.SemaphoreType.DMA,
    ],
)

_EA = _NS * _NCH_A * _CH


def _dot(a, b):
    return jnp.dot(a, b, precision=lax.Precision.HIGHEST,
                   preferred_element_type=_f32)


def _layer1_body(ps, pc, xin, wl, bl, wr, out):
    p = ps[...]
    c = pc[...]
    cnt = c[0, :, 0:1] + c[1, :, 0:1]
    mean = (p[0] + p[1]) / jnp.maximum(cnt, 1.0)
    h = _dot(mean, wl[...]) + bl[...] + _dot(xin[...], wr[...])
    out[...] = jnp.maximum(h, 0.0)


def _layer2_body(ps, pc, hin, wl, bl, wr, wlin, blin, out):
    p = ps[...]
    c = pc[...]
    cnt = c[0, :, 0:1] + c[1, :, 0:1]
    mean = (p[0] + p[1]) / jnp.maximum(cnt, 1.0)
    h1 = _dot(mean, wl[...]) + bl[...] + _dot(hin[...], wr[...])
    out[...] = _dot(h1, wlin[...]) + blin[...]


_BS = 2000
_GRID = (_N // _BS,)


def _layer1(psum, pcnt, x, Wl, bl, Wr):
    return pl.pallas_call(
        _layer1_body,
        grid=_GRID,
        in_specs=[
            pl.BlockSpec((_NC, _BS, _D), lambda i: (0, i, 0)),
            pl.BlockSpec((_NC, _BS, _D), lambda i: (0, i, 0)),
            pl.BlockSpec((_BS, _D), lambda i: (i, 0)),
            pl.BlockSpec((_D, _H), lambda i: (0, 0)),
            pl.BlockSpec((1, _H), lambda i: (0, 0)),
            pl.BlockSpec((_D, _H), lambda i: (0, 0)),
        ],
        out_specs=pl.BlockSpec((_BS, _H), lambda i: (i, 0)),
        out_shape=jax.ShapeDtypeStruct((_N, _H), _f32),
    )(psum, pcnt, x, Wl, bl, Wr)


def _layer2(psum, pcnt, h, Wl, bl, Wr, Wlin, blin):
    return pl.pallas_call(
        _layer2_body,
        grid=_GRID,
        in_specs=[
            pl.BlockSpec((_NC, _BS, _H), lambda i: (0, i, 0)),
            pl.BlockSpec((_NC, _BS, _D), lambda i: (0, i, 0)),
            pl.BlockSpec((_BS, _H), lambda i: (i, 0)),
            pl.BlockSpec((_H, _H), lambda i: (0, 0)),
            pl.BlockSpec((1, _H), lambda i: (0, 0)),
            pl.BlockSpec((_H, _H), lambda i: (0, 0)),
            pl.BlockSpec((_H, _C), lambda i: (0, 0)),
            pl.BlockSpec((1, _C), lambda i: (0, 0)),
        ],
        out_specs=pl.BlockSpec((_BS, _C), lambda i: (i, 0)),
        out_shape=jax.ShapeDtypeStruct((_N, _C), _f32),
    )(psum, pcnt, h, Wl, bl, Wr, Wlin, blin)


def kernel(x, edge_index, relations, Wl0, bl0, Wr0, Wl1, bl1, Wr1, Wlin, blin):
    del relations
    npad = _EP - _E
    iota = lax.iota(jnp.int32, npad)
    src = jnp.concatenate([edge_index[0], iota % _N])
    dst = jnp.concatenate([edge_index[1], _N + iota % (_NP - _N)])
    src3 = src.reshape(_NW, _NCHUNK, _CH)
    dst3 = dst.reshape(_NW, _NCHUNK, _CH)
    srcA = src[:_EA].reshape(_NS, _NCH_A, _CH)
    dstA = dst[:_EA].reshape(_NS, _NCH_A, _CH)
    srcB = src[_EA:].reshape(_NS, _NCH_B, _CH)
    dstB = dst[_EA:].reshape(_NS, _NCH_B, _CH)
    zrows = jnp.zeros((_RPT, _D), _f32)
    ones = jnp.ones((_CH, _D), _f32)

    pcnt = _cnt_pass(dst3, zrows, ones)
    psum0 = _seg_sum(x, srcA, dstA, srcB, dstB, zrows)
    h = _layer1(psum0, pcnt, x, Wl0, bl0.reshape(1, _H), Wr0)
    psum1 = _seg_sum(h, srcA, dstA, srcB, dstB, zrows)
    out = _layer2(psum1, pcnt, h, Wl1, bl1.reshape(1, _H), Wr1,
                  Wlin, blin.reshape(1, _C))
    return out


# speedup vs baseline: 8.5538x; 1.0087x over previous
import functools

import jax
import jax.numpy as jnp
from jax import lax
from jax.experimental import pallas as pl
from jax.experimental.pallas import tpu as pltpu
from jax.experimental.pallas import tpu_sc as plsc

_N, _E, _D, _H, _C = 10000, 320000, 128, 128, 41
_NC, _NS = 2, 16
_NW = _NC * _NS
_CH = 128
_NCHUNK = 80
_EPW = _NCHUNK * _CH
_EP = _NW * _EPW
_BLK = 8
_NBLK = _NCHUNK // _BLK
_NCH_A = 80
_NCH_B = 2 * _NCHUNK - _NCH_A
_NB = 2
_AH = 1
_NP = 10112
_RPT = _NP // _NS

_f32 = jnp.float32

_mesh = plsc.VectorSubcoreMesh(core_axis_name="c", subcore_axis_name="s")


def _cnt_body(dst3, zrows, ones_hbm, pcnt, cnt_sh, didx_blk, ones_v, ssem):
    cid = lax.axis_index("c")
    sid = lax.axis_index("s")
    wid = sid * _NC + cid
    rbase = sid * _RPT

    pltpu.sync_copy(zrows, cnt_sh.at[pl.ds(rbase, _RPT)])
    pltpu.sync_copy(ones_hbm, ones_v)
    plsc.subcore_barrier()

    def blk_step(bk, carry):
        row0 = pl.multiple_of(bk * _BLK, _BLK)
        pltpu.sync_copy(dst3.at[wid, pl.ds(row0, _BLK)], didx_blk)
        hs = [pltpu.async_copy(ones_v, cnt_sh.at[didx_blk.at[b]], ssem,
                               add=True)
              for b in range(_BLK)]
        for h in hs:
            h.wait()
        return carry

    lax.fori_loop(0, _NBLK, blk_step, 0)
    plsc.subcore_barrier()

    pltpu.sync_copy(cnt_sh.at[pl.ds(rbase, _RPT)],
                    pcnt.at[cid, pl.ds(rbase, _RPT)])


def _seg_body(feat, srcA, dstA, srcB, dstB, zrows, psum,
              acc_sh, idx_blk, didx_blk, rows2, gsem, ssem):
    cid = lax.axis_index("c")
    sid = lax.axis_index("s")
    rbase = sid * _RPT

    pltpu.sync_copy(zrows, acc_sh.at[pl.ds(rbase, _RPT)])
    plsc.subcore_barrier()

    def blk_step(src3, dst3, bk):
        row0 = pl.multiple_of(bk * _BLK, _BLK)
        pltpu.sync_copy(src3.at[sid, pl.ds(row0, _BLK)], idx_blk)
        pltpu.sync_copy(dst3.at[sid, pl.ds(row0, _BLK)], didx_blk)
        gh = {b: pltpu.async_copy(feat.at[idx_blk.at[b]], rows2.at[b % _NB],
                                  gsem)
              for b in range(_AH)}
        sh = {}
        for b in range(_BLK):
            gh[b].wait()
            nb = b + _AH
            if nb < _BLK:
                if nb >= _NB:
                    sh[nb - _NB].wait()
                gh[nb] = pltpu.async_copy(feat.at[idx_blk.at[nb]],
                                          rows2.at[nb % _NB], gsem)
            sh[b] = pltpu.async_copy(rows2.at[b % _NB],
                                     acc_sh.at[didx_blk.at[b]], ssem,
                                     add=True)
        for b in range(max(0, _BLK - _NB), _BLK):
            sh[b].wait()

    @pl.when(cid == 0)
    def _():
        lax.fori_loop(0, _NCH_A // _BLK,
                      lambda bk, c: (blk_step(srcA, dstA, bk), c)[1], 0)

    @pl.when(cid == 1)
    def _():
        lax.fori_loop(0, _NCH_B // _BLK,
                      lambda bk, c: (blk_step(srcB, dstB, bk), c)[1], 0)

    plsc.subcore_barrier()

    pltpu.sync_copy(acc_sh.at[pl.ds(rbase, _RPT)],
                    psum.at[cid, pl.ds(rbase, _RPT)])


_cnt_pass = pl.kernel(
    _cnt_body,
    out_type=jax.ShapeDtypeStruct((_NC, _NP, _D), _f32),
    mesh=_mesh,
    scratch_types=[
        pltpu.VMEM_SHARED((_NP, _D), _f32),
        pltpu.VMEM((_BLK, _CH), jnp.int32),
        pltpu.VMEM((_CH, _D), _f32),
        pltpu.SemaphoreType.DMA,
    ],
)

_seg_sum = pl.kernel(
    _seg_body,
    out_type=jax.ShapeDtypeStruct((_NC, _NP, _D), _f32),
    mesh=_mesh,
    scratch_types=[
        pltpu.VMEM_SHARED((_NP, _D), _f32),
        pltpu.VMEM((_BLK, _CH), jnp.int32),
        pltpu.VMEM((_BLK, _CH), jnp.int32),
        pltpu.VMEM((_NB, _CH, _D), _f32),
        pltpu.SemaphoreType.DMA,
        pltpu.SemaphoreType.DMA,
    ],
)

_EA = _NS * _NCH_A * _CH


def _dot(a, b):
    return jnp.dot(a, b, precision=lax.Precision.HIGHEST,
                   preferred_element_type=_f32)


def _xw_body(xin, wr, out):
    out[...] = _dot(xin[...], wr[...])


def _combine1_body(ps, pc, xw, wl, bl, out):
    p = ps[...]
    c = pc[...]
    cnt = c[0, :, 0:1] + c[1, :, 0:1]
    mean = (p[0] + p[1]) / jnp.maximum(cnt, 1.0)
    h = _dot(mean, wl[...]) + bl[...] + xw[...]
    out[...] = jnp.maximum(h, 0.0)


def _combine2_body(ps, pc, hw, wl, bl, wlin, blin, out):
    p = ps[...]
    c = pc[...]
    cnt = c[0, :, 0:1] + c[1, :, 0:1]
    mean = (p[0] + p[1]) / jnp.maximum(cnt, 1.0)
    h1 = _dot(mean, wl[...]) + bl[...] + hw[...]
    out[...] = _dot(h1, wlin[...]) + blin[...]


_BS = 2000
_GRID = (_N // _BS,)


def _xw(x, Wr):
    return pl.pallas_call(
        _xw_body,
        grid=_GRID,
        in_specs=[
            pl.BlockSpec((_BS, _D), lambda i: (i, 0)),
            pl.BlockSpec((_D, _H), lambda i: (0, 0)),
        ],
        out_specs=pl.BlockSpec((_BS, _H), lambda i: (i, 0)),
        out_shape=jax.ShapeDtypeStruct((_N, _H), _f32),
    )(x, Wr)


def _combine1(psum, pcnt, xw, Wl, bl):
    return pl.pallas_call(
        _combine1_body,
        grid=_GRID,
        in_specs=[
            pl.BlockSpec((_NC, _BS, _D), lambda i: (0, i, 0)),
            pl.BlockSpec((_NC, _BS, _D), lambda i: (0, i, 0)),
            pl.BlockSpec((_BS, _H), lambda i: (i, 0)),
            pl.BlockSpec((_D, _H), lambda i: (0, 0)),
            pl.BlockSpec((1, _H), lambda i: (0, 0)),
        ],
        out_specs=pl.BlockSpec((_BS, _H), lambda i: (i, 0)),
        out_shape=jax.ShapeDtypeStruct((_N, _H), _f32),
    )(psum, pcnt, xw, Wl, bl)


def _combine2(psum, pcnt, hw, Wl, bl, Wlin, blin):
    return pl.pallas_call(
        _combine2_body,
        grid=_GRID,
        in_specs=[
            pl.BlockSpec((_NC, _BS, _H), lambda i: (0, i, 0)),
            pl.BlockSpec((_NC, _BS, _D), lambda i: (0, i, 0)),
            pl.BlockSpec((_BS, _H), lambda i: (i, 0)),
            pl.BlockSpec((_H, _H), lambda i: (0, 0)),
            pl.BlockSpec((1, _H), lambda i: (0, 0)),
            pl.BlockSpec((_H, _C), lambda i: (0, 0)),
            pl.BlockSpec((1, _C), lambda i: (0, 0)),
        ],
        out_specs=pl.BlockSpec((_BS, _C), lambda i: (i, 0)),
        out_shape=jax.ShapeDtypeStruct((_N, _C), _f32),
    )(psum, pcnt, hw, Wl, bl, Wlin, blin)


def kernel(x, edge_index, relations, Wl0, bl0, Wr0, Wl1, bl1, Wr1, Wlin, blin):
    del relations
    npad = _EP - _E
    iota = lax.iota(jnp.int32, npad)
    src = jnp.concatenate([edge_index[0], iota % _N])
    dst = jnp.concatenate([edge_index[1], _N + iota % (_NP - _N)])
    src3 = src.reshape(_NW, _NCHUNK, _CH)
    dst3 = dst.reshape(_NW, _NCHUNK, _CH)
    srcA = src[:_EA].reshape(_NS, _NCH_A, _CH)
    dstA = dst[:_EA].reshape(_NS, _NCH_A, _CH)
    srcB = src[_EA:].reshape(_NS, _NCH_B, _CH)
    dstB = dst[_EA:].reshape(_NS, _NCH_B, _CH)
    zrows = jnp.zeros((_RPT, _D), _f32)
    ones = jnp.ones((_CH, _D), _f32)

    pcnt = _cnt_pass(dst3, zrows, ones)
    xw = _xw(x, Wr0)
    psum0 = _seg_sum(x, srcA, dstA, srcB, dstB, zrows)
    h = _combine1(psum0, pcnt, xw, Wl0, bl0.reshape(1, _H))
    hw = _xw(h, Wr1)
    psum1 = _seg_sum(h, srcA, dstA, srcB, dstB, zrows)
    out = _combine2(psum1, pcnt, hw, Wl1, bl1.reshape(1, _H),
                    Wlin, blin.reshape(1, _C))
    return out
